# baseline (speedup 1.0000x reference)
import functools

import jax
import jax.numpy as jnp
from jax import lax
from jax.experimental import pallas as pl
from jax.experimental.pallas import tpu as pltpu
from jax.experimental.pallas import tpu_sc as plsc

VOCAB = 100000
D_MODEL = 1024
BATCH = 4
SEQ = 4096
SCALE = 32.0

NC = 2
NS = 16
NW = NC * NS
LANES = 16

POS_PER_W = SEQ // NW
PC = 32
NCHUNK = POS_PER_W // PC
VPR = D_MODEL // LANES


def _sc_body(x_hbm, pe_hbm, table_hbm, out_hbm, idx_v, pe_v, tab_v, sem):
    wid = lax.axis_index("s") * NC + lax.axis_index("c")

    def chunk_body(c, _):
        pos0 = wid * POS_PER_W + c * PC
        pltpu.sync_copy(pe_hbm.at[pl.ds(pos0, PC)], pe_v)

        def batch_body(b, _):
            row0 = b * SEQ + pos0
            pltpu.sync_copy(x_hbm.at[pl.ds(row0, PC)], idx_v)
            pltpu.async_copy(table_hbm.at[idx_v], tab_v, sem).wait()

            def row_body(r, _):
                def col_body(j, _):
                    sl = pl.ds(j * LANES, LANES)
                    tab_v[r, sl] = tab_v[r, sl] * SCALE + pe_v[r, sl]
                    return _
                return lax.fori_loop(0, VPR, col_body, _, unroll=4)

            lax.fori_loop(0, PC, row_body, None)
            pltpu.sync_copy(tab_v, out_hbm.at[pl.ds(row0, PC)])
            return _

        lax.fori_loop(0, BATCH, batch_body, None)
        return _

    lax.fori_loop(0, NCHUNK, chunk_body, None)


@jax.jit
def _embed(x_flat, table, pe_flat):
    mesh = plsc.VectorSubcoreMesh(core_axis_name="c", subcore_axis_name="s")
    out = pl.kernel(
        _sc_body,
        out_type=jax.ShapeDtypeStruct((BATCH * SEQ, D_MODEL), jnp.float32),
        mesh=mesh,
        scratch_types=[
            pltpu.VMEM((PC,), jnp.int32),
            pltpu.VMEM((PC, D_MODEL), jnp.float32),
            pltpu.VMEM((PC, D_MODEL), jnp.float32),
            pltpu.SemaphoreType.DMA,
        ],
    )(x_flat, pe_flat, table)
    return out


def kernel(x, table, pe):
    x_flat = x.reshape(BATCH * SEQ).astype(jnp.int32)
    pe_flat = pe.reshape(-1, D_MODEL)[:SEQ]
    out = _embed(x_flat, table, pe_flat)
    return out.reshape(BATCH, SEQ, D_MODEL)


# speedup vs baseline: 1.2437x; 1.2437x over previous
import jax
import jax.numpy as jnp
from jax import lax
from jax.experimental import pallas as pl
from jax.experimental.pallas import tpu as pltpu
from jax.experimental.pallas import tpu_sc as plsc

VOCAB = 100000
D_MODEL = 1024
BATCH = 4
SEQ = 4096
SCALE = 32.0

NC = 2
NS = 16
NW = NC * NS
LANES = 16

POS_PER_W = SEQ // NW
PC = 32
NCHUNK = POS_PER_W // PC
NSTEPS = NCHUNK * BATCH
VPR = D_MODEL // LANES


def _sc_body(x_hbm, pe_hbm, table_hbm, out_hbm,
             idxa, pe_v, tb0, tb1, g0, g1, o0, o1):
    wid = lax.axis_index("s") * NC + lax.axis_index("c")
    pos_base = wid * POS_PER_W
    tb, g, o = (tb0, tb1), (g0, g1), (o0, o1)

    for b in range(BATCH):
        pltpu.sync_copy(x_hbm.at[pl.ds(b * SEQ + pos_base, POS_PER_W)],
                        idxa.at[b])
    pltpu.sync_copy(pe_hbm.at[pl.ds(pos_base, PC)], pe_v)

    def start_gather(step, buf, sem):
        c, b = divmod(step, BATCH)
        idx_ref = idxa.at[b, pl.ds(c * PC, PC)]
        return pltpu.async_copy(table_hbm.at[idx_ref], buf, sem)

    pending = {("g", 0): start_gather(0, tb[0], g[0])}
    for i in range(NSTEPS):
        p = i % 2
        c, b = divmod(i, BATCH)
        nxt = i + 1
        if nxt < NSTEPS:
            pn = nxt % 2
            if nxt >= 2:
                pending.pop(("o", nxt - 2)).wait()
            pending[("g", nxt)] = start_gather(nxt, tb[pn], g[pn])
        pending.pop(("g", i)).wait()
        if i > 0 and b == 0:
            pltpu.sync_copy(pe_hbm.at[pl.ds(pos_base + c * PC, PC)], pe_v)

        buf = tb[p]

        def row_body(r, carry, buf=buf):
            def col_body(j, carry2):
                sl = pl.ds(j * LANES, LANES)
                buf[r, sl] = buf[r, sl] * SCALE + pe_v[r, sl]
                return carry2
            return lax.fori_loop(0, VPR, col_body, carry, unroll=4)

        lax.fori_loop(0, PC, row_body, None)
        row0 = b * SEQ + pos_base + c * PC
        pending[("o", i)] = pltpu.async_copy(
            buf, out_hbm.at[pl.ds(row0, PC)], o[p])

    pending.pop(("o", NSTEPS - 2)).wait()
    pending.pop(("o", NSTEPS - 1)).wait()


@jax.jit
def _embed(x_flat, table, pe_flat):
    mesh = plsc.VectorSubcoreMesh(core_axis_name="c", subcore_axis_name="s")
    out = pl.kernel(
        _sc_body,
        out_type=jax.ShapeDtypeStruct((BATCH * SEQ, D_MODEL), jnp.float32),
        mesh=mesh,
        scratch_types=[
            pltpu.VMEM((BATCH, POS_PER_W), jnp.int32),
            pltpu.VMEM((PC, D_MODEL), jnp.float32),
            pltpu.VMEM((PC, D_MODEL), jnp.float32),
            pltpu.VMEM((PC, D_MODEL), jnp.float32),
            pltpu.SemaphoreType.DMA,
            pltpu.SemaphoreType.DMA,
            pltpu.SemaphoreType.DMA,
            pltpu.SemaphoreType.DMA,
        ],
    )(x_flat, pe_flat, table)
    return out


def kernel(x, table, pe):
    x_flat = x.reshape(BATCH * SEQ).astype(jnp.int32)
    pe_flat = pe.reshape(-1, D_MODEL)[:SEQ]
    out = _embed(x_flat, table, pe_flat)
    return out.reshape(BATCH, SEQ, D_MODEL)


# speedup vs baseline: 1.3780x; 1.1081x over previous
import jax
import jax.numpy as jnp
from jax import lax
from jax.experimental import pallas as pl
from jax.experimental.pallas import tpu as pltpu
from jax.experimental.pallas import tpu_sc as plsc

VOCAB = 100000
D_MODEL = 1024
BATCH = 4
SEQ = 4096
SCALE = 32.0

NC = 2
NS = 16
NW = NC * NS
LANES = 16

POS_PER_W = SEQ // NW
PC = 16
NCHUNK = POS_PER_W // PC
NSTEPS = NCHUNK * BATCH
NBUF = 4
VPR = D_MODEL // LANES


def _sc_body(x_hbm, pe_hbm, table_hbm, out_hbm,
             idxa, pe0, pe1, tb0, tb1, tb2, tb3,
             g0, g1, g2, g3, o0, o1, o2, o3, q0, q1):
    wid = lax.axis_index("s") * NC + lax.axis_index("c")
    pos_base = wid * POS_PER_W
    tb, g, o = (tb0, tb1, tb2, tb3), (g0, g1, g2, g3), (o0, o1, o2, o3)
    pe_v, q = (pe0, pe1), (q0, q1)

    for b in range(BATCH):
        pltpu.sync_copy(x_hbm.at[pl.ds(b * SEQ + pos_base, POS_PER_W)],
                        idxa.at[b])

    def start_gather(step):
        c, b = divmod(step, BATCH)
        idx_ref = idxa.at[b, pl.ds(c * PC, PC)]
        return pltpu.async_copy(table_hbm.at[idx_ref], tb[step % NBUF],
                                g[step % NBUF])

    def start_pe(c):
        return pltpu.async_copy(pe_hbm.at[pl.ds(pos_base + c * PC, PC)],
                                pe_v[c % 2], q[c % 2])

    pending = {("q", 0): start_pe(0), ("q", 1): start_pe(1)}
    for j in range(NBUF - 1):
        pending[("g", j)] = start_gather(j)

    for i in range(NSTEPS):
        p = i % NBUF
        c, b = divmod(i, BATCH)
        jn = i + NBUF - 1
        if jn < NSTEPS:
            if jn >= NBUF:
                pending.pop(("o", jn - NBUF)).wait()
            pending[("g", jn)] = start_gather(jn)
        pending.pop(("g", i)).wait()
        if b == 0:
            pending.pop(("q", c)).wait()

        buf, pe_b = tb[p], pe_v[c % 2]

        def row_body(r, carry, buf=buf, pe_b=pe_b):
            def col_body(k, carry2):
                sl = pl.ds(k * LANES, LANES)
                buf[r, sl] = buf[r, sl] * SCALE + pe_b[r, sl]
                return carry2
            return lax.fori_loop(0, VPR, col_body, carry, unroll=4)

        lax.fori_loop(0, PC, row_body, None)
        if b == BATCH - 1 and c + 2 < NCHUNK:
            pending[("q", c + 2)] = start_pe(c + 2)
        row0 = b * SEQ + pos_base + c * PC
        pending[("o", i)] = pltpu.async_copy(
            buf, out_hbm.at[pl.ds(row0, PC)], o[p])

    for i in range(NSTEPS - NBUF, NSTEPS):
        pending.pop(("o", i)).wait()


@jax.jit
def _embed(x_flat, table, pe_flat):
    mesh = plsc.VectorSubcoreMesh(core_axis_name="c", subcore_axis_name="s")
    out = pl.kernel(
        _sc_body,
        out_type=jax.ShapeDtypeStruct((BATCH * SEQ, D_MODEL), jnp.float32),
        mesh=mesh,
        scratch_types=(
            [pltpu.VMEM((BATCH, POS_PER_W), jnp.int32)]
            + [pltpu.VMEM((PC, D_MODEL), jnp.float32) for _ in range(2 + NBUF)]
            + [pltpu.SemaphoreType.DMA for _ in range(2 * NBUF + 2)]
        ),
    )(x_flat, pe_flat, table)
    return out


def kernel(x, table, pe):
    x_flat = x.reshape(BATCH * SEQ).astype(jnp.int32)
    pe_flat = pe.reshape(-1, D_MODEL)[:SEQ]
    out = _embed(x_flat, table, pe_flat)
    return out.reshape(BATCH, SEQ, D_MODEL)
